# baseline (speedup 1.0000x reference)
import functools

import numpy as np
import jax
import jax.numpy as jnp
from jax import lax
from jax.experimental import pallas as pl
from jax.experimental.pallas import tpu as pltpu
from jax.experimental.pallas import tpu_sc as plsc

S = 250
B = 128
K = 1000
S_PAD = 256


def _np_threefry2x32(k1, k2, x0, x1):
    def rotl(x, d):
        return ((x << np.uint32(d)) | (x >> np.uint32(32 - d))).astype(np.uint32)

    ks0, ks1 = np.uint32(k1), np.uint32(k2)
    ks2 = np.uint32(ks0 ^ ks1 ^ np.uint32(0x1BD11BDA))
    rots = ([13, 15, 26, 6], [17, 29, 16, 24])
    sched = [(ks1, ks2, 1), (ks2, ks0, 2), (ks0, ks1, 3), (ks1, ks2, 4), (ks2, ks0, 5)]
    x0 = np.uint32(x0 + ks0)
    x1 = np.uint32(x1 + ks1)
    for i, (a, b, c) in enumerate(sched):
        for r in rots[i % 2]:
            x0 = np.uint32(x0 + x1)
            x1 = rotl(x1, r)
            x1 = np.uint32(x0 ^ x1)
        x0 = np.uint32(x0 + a)
        x1 = np.uint32(x1 + b + np.uint32(c))
    return x0, x1


_GK1, _GK2 = _np_threefry2x32(0, 0, np.uint32(0), np.uint32(123))
_KS = (int(_GK1), int(_GK2), int(np.uint32(_GK1 ^ _GK2 ^ np.uint32(0x1BD11BDA))))


def _i32(v):
    return jnp.int32(np.int32(np.uint32(v)))


def _rotl(x, d):
    return jax.lax.shift_left(x, jnp.int32(d)) | jax.lax.shift_right_logical(
        x, jnp.int32(32 - d)
    )


def _threefry_hash(x0, x1):
    rots = ([13, 15, 26, 6], [17, 29, 16, 24])
    sched = [
        (_KS[1], _KS[2], 1),
        (_KS[2], _KS[0], 2),
        (_KS[0], _KS[1], 3),
        (_KS[1], _KS[2], 4),
        (_KS[2], _KS[0], 5),
    ]
    x0 = x0 + _i32(_KS[0])
    x1 = x1 + _i32(_KS[1])
    for i, (a, b, c) in enumerate(sched):
        for r in rots[i % 2]:
            x0 = x0 + x1
            x1 = _rotl(x1, r)
            x1 = x0 ^ x1
        x0 = x0 + _i32(a)
        x1 = x1 + _i32(np.uint32(b) + np.uint32(c))
    return x0, x1


_TINY = np.float32(np.finfo(np.float32).tiny)
_SPAN = np.float32(np.float32(1.0) - _TINY)

CHUNK = 128
N_FULL = 7
TAIL = K - N_FULL * CHUNK
S_PER_STEP = 4
GRID = S_PAD // S_PER_STEP

_NEG_HUGE = np.float32(-1e30)


def _chunk_update(s, base_k, rows, logits_t_ref, classes_ref, carry):
    m, cv = carry
    iota_r = jax.lax.broadcasted_iota(jnp.int32, (rows, B), 0)
    iota_b = jax.lax.broadcasted_iota(jnp.int32, (rows, B), 1)
    flat = (s * (B * K) + base_k) + (iota_b * K + iota_r)
    h0, h1 = _threefry_hash(jnp.zeros((rows, B), jnp.int32), flat)
    bits = h0 ^ h1
    float_bits = jax.lax.shift_right_logical(bits, jnp.int32(9)) | _i32(0x3F800000)
    floats = jax.lax.bitcast_convert_type(float_bits, jnp.float32) - jnp.float32(1.0)
    u = jnp.maximum(_TINY, floats)
    g = -jnp.log(-jnp.log(u))
    val = logits_t_ref[pl.ds(base_k, rows), :] + g
    cm = jnp.max(val, axis=0, keepdims=True)
    ic = jnp.min(jnp.where(val == cm, iota_r, rows), axis=0, keepdims=True)
    cls = classes_ref[pl.ds(base_k, rows), :]
    ccls = jnp.sum(jnp.where(iota_r == ic, cls, jnp.float32(0.0)), axis=0,
                   keepdims=True)
    upd = cm > m
    return jnp.where(upd, cm, m), jnp.where(upd, ccls, cv)


def _sampler_kernel(logits_t_ref, classes_ref, out_ref):
    step = pl.program_id(0)

    def sample_body(sl, _):
        s = step * S_PER_STEP + sl

        def chunk_body(c, carry):
            return _chunk_update(
                s, c * CHUNK, CHUNK, logits_t_ref, classes_ref, carry
            )

        m0 = jnp.full((1, B), -jnp.inf, jnp.float32)
        bc0 = jnp.zeros((1, B), jnp.float32)
        carry = jax.lax.fori_loop(0, N_FULL, chunk_body, (m0, bc0), unroll=7)
        _, bc = _chunk_update(
            s, N_FULL * CHUNK, TAIL, logits_t_ref, classes_ref, carry
        )
        out_ref[pl.ds(sl, 1), 0, :] = bc
        return 0

    jax.lax.fori_loop(0, S_PER_STEP, sample_body, 0, unroll=S_PER_STEP)


def kernel(logits, classes):
    logits_t = logits.T
    classes_col = classes.reshape(K, 1)
    out = pl.pallas_call(
        _sampler_kernel,
        grid=(GRID,),
        in_specs=[
            pl.BlockSpec((K, B), lambda i: (0, 0)),
            pl.BlockSpec((K, 1), lambda i: (0, 0)),
        ],
        out_specs=pl.BlockSpec((S_PER_STEP, 1, B), lambda i: (i, 0, 0)),
        out_shape=jax.ShapeDtypeStruct((S_PAD, 1, B), jnp.float32),
    )(logits_t, classes_col)
    return out[:S, 0, :].T.reshape(B, 1, 1, S)


# speedup vs baseline: 1.0001x; 1.0001x over previous
import functools

import numpy as np
import jax
import jax.numpy as jnp
from jax import lax
from jax.experimental import pallas as pl
from jax.experimental.pallas import tpu as pltpu
from jax.experimental.pallas import tpu_sc as plsc

S = 250
B = 128
K = 1000
S_PAD = 256


def _np_threefry2x32(k1, k2, x0, x1):
    def rotl(x, d):
        return ((x << np.uint32(d)) | (x >> np.uint32(32 - d))).astype(np.uint32)

    ks0, ks1 = np.uint32(k1), np.uint32(k2)
    ks2 = np.uint32(ks0 ^ ks1 ^ np.uint32(0x1BD11BDA))
    rots = ([13, 15, 26, 6], [17, 29, 16, 24])
    sched = [(ks1, ks2, 1), (ks2, ks0, 2), (ks0, ks1, 3), (ks1, ks2, 4), (ks2, ks0, 5)]
    x0 = np.uint32(x0 + ks0)
    x1 = np.uint32(x1 + ks1)
    for i, (a, b, c) in enumerate(sched):
        for r in rots[i % 2]:
            x0 = np.uint32(x0 + x1)
            x1 = rotl(x1, r)
            x1 = np.uint32(x0 ^ x1)
        x0 = np.uint32(x0 + a)
        x1 = np.uint32(x1 + b + np.uint32(c))
    return x0, x1


_GK1, _GK2 = _np_threefry2x32(0, 0, np.uint32(0), np.uint32(123))
_KS = (int(_GK1), int(_GK2), int(np.uint32(_GK1 ^ _GK2 ^ np.uint32(0x1BD11BDA))))


def _i32(v):
    return jnp.int32(np.int32(np.uint32(v)))


def _rotl(x, d):
    return jax.lax.shift_left(x, jnp.int32(d)) | jax.lax.shift_right_logical(
        x, jnp.int32(32 - d)
    )


def _threefry_hash(x0, x1):
    rots = ([13, 15, 26, 6], [17, 29, 16, 24])
    sched = [
        (_KS[1], _KS[2], 1),
        (_KS[2], _KS[0], 2),
        (_KS[0], _KS[1], 3),
        (_KS[1], _KS[2], 4),
        (_KS[2], _KS[0], 5),
    ]
    x0 = x0 + _i32(_KS[0])
    x1 = x1 + _i32(_KS[1])
    for i, (a, b, c) in enumerate(sched):
        for r in rots[i % 2]:
            x0 = x0 + x1
            x1 = _rotl(x1, r)
            x1 = x0 ^ x1
        x0 = x0 + _i32(a)
        x1 = x1 + _i32(np.uint32(b) + np.uint32(c))
    return x0, x1


_TINY = np.float32(np.finfo(np.float32).tiny)
_SPAN = np.float32(np.float32(1.0) - _TINY)

CHUNK = 128
N_FULL = 7
TAIL = K - N_FULL * CHUNK
S_PER_STEP = 4
GRID = S_PAD // S_PER_STEP

_NEG_HUGE = np.float32(-1e30)


def _chunk_update(s, base_k, rows, logits_t_ref, classes_ref, carry):
    m, cv = carry
    iota_r = jax.lax.broadcasted_iota(jnp.int32, (rows, B), 0)
    iota_b = jax.lax.broadcasted_iota(jnp.int32, (rows, B), 1)
    flat = (s * (B * K) + base_k) + (iota_b * K + iota_r)
    h0, h1 = _threefry_hash(jnp.zeros((rows, B), jnp.int32), flat)
    bits = h0 ^ h1
    float_bits = jax.lax.shift_right_logical(bits, jnp.int32(9)) | _i32(0x3F800000)
    floats = jax.lax.bitcast_convert_type(float_bits, jnp.float32) - jnp.float32(1.0)
    u = jnp.maximum(_TINY, floats)
    g = -jnp.log(-jnp.log(u))
    val = logits_t_ref[pl.ds(base_k, rows), :] + g
    cm = jnp.max(val, axis=0, keepdims=True)
    ic = jnp.min(jnp.where(val == cm, iota_r, rows), axis=0, keepdims=True)
    cls = classes_ref[pl.ds(base_k, rows), :]
    ccls = jnp.sum(jnp.where(iota_r == ic, cls, jnp.float32(0.0)), axis=0,
                   keepdims=True)
    upd = cm > m
    return jnp.where(upd, cm, m), jnp.where(upd, ccls, cv)


def _sampler_kernel(logits_t_ref, classes_ref, out_ref):
    step = pl.program_id(0)

    def sample_body(sl, _):
        s = step * S_PER_STEP + sl

        def chunk_body(c, carry):
            return _chunk_update(
                s, c * CHUNK, CHUNK, logits_t_ref, classes_ref, carry
            )

        m0 = jnp.full((1, B), -jnp.inf, jnp.float32)
        bc0 = jnp.zeros((1, B), jnp.float32)
        carry = jax.lax.fori_loop(0, N_FULL, chunk_body, (m0, bc0), unroll=7)
        _, bc = _chunk_update(
            s, N_FULL * CHUNK, TAIL, logits_t_ref, classes_ref, carry
        )
        out_ref[pl.ds(sl, 1), 0, :] = bc
        return 0

    jax.lax.fori_loop(0, S_PER_STEP, sample_body, 0, unroll=S_PER_STEP)


def kernel(logits, classes):
    logits_t = logits.T
    classes_col = classes.reshape(K, 1)
    out = pl.pallas_call(
        _sampler_kernel,
        grid=(GRID,),
        in_specs=[
            pl.BlockSpec((K, B), lambda i: (0, 0)),
            pl.BlockSpec((K, 1), lambda i: (0, 0)),
        ],
        out_specs=pl.BlockSpec((S_PER_STEP, 1, B), lambda i: (i, 0, 0)),
        out_shape=jax.ShapeDtypeStruct((S_PAD, 1, B), jnp.float32),
        compiler_params=pltpu.CompilerParams(
            dimension_semantics=("parallel",),
        ),
    )(logits_t, classes_col)
    return out[:S, 0, :].T.reshape(B, 1, 1, S)


# speedup vs baseline: 1.0257x; 1.0256x over previous
import functools

import numpy as np
import jax
import jax.numpy as jnp
from jax import lax
from jax.experimental import pallas as pl
from jax.experimental.pallas import tpu as pltpu
from jax.experimental.pallas import tpu_sc as plsc

S = 250
B = 128
K = 1000
S_PAD = 250


def _np_threefry2x32(k1, k2, x0, x1):
    def rotl(x, d):
        return ((x << np.uint32(d)) | (x >> np.uint32(32 - d))).astype(np.uint32)

    ks0, ks1 = np.uint32(k1), np.uint32(k2)
    ks2 = np.uint32(ks0 ^ ks1 ^ np.uint32(0x1BD11BDA))
    rots = ([13, 15, 26, 6], [17, 29, 16, 24])
    sched = [(ks1, ks2, 1), (ks2, ks0, 2), (ks0, ks1, 3), (ks1, ks2, 4), (ks2, ks0, 5)]
    x0 = np.uint32(x0 + ks0)
    x1 = np.uint32(x1 + ks1)
    for i, (a, b, c) in enumerate(sched):
        for r in rots[i % 2]:
            x0 = np.uint32(x0 + x1)
            x1 = rotl(x1, r)
            x1 = np.uint32(x0 ^ x1)
        x0 = np.uint32(x0 + a)
        x1 = np.uint32(x1 + b + np.uint32(c))
    return x0, x1


_GK1, _GK2 = _np_threefry2x32(0, 0, np.uint32(0), np.uint32(123))
_KS = (int(_GK1), int(_GK2), int(np.uint32(_GK1 ^ _GK2 ^ np.uint32(0x1BD11BDA))))


def _i32(v):
    return jnp.int32(np.int32(np.uint32(v)))


def _rotl(x, d):
    return jax.lax.shift_left(x, jnp.int32(d)) | jax.lax.shift_right_logical(
        x, jnp.int32(32 - d)
    )


def _threefry_hash(x0, x1):
    rots = ([13, 15, 26, 6], [17, 29, 16, 24])
    sched = [
        (_KS[1], _KS[2], 1),
        (_KS[2], _KS[0], 2),
        (_KS[0], _KS[1], 3),
        (_KS[1], _KS[2], 4),
        (_KS[2], _KS[0], 5),
    ]
    x0 = x0 + _i32(_KS[0])
    x1 = x1 + _i32(_KS[1])
    for i, (a, b, c) in enumerate(sched):
        for r in rots[i % 2]:
            x0 = x0 + x1
            x1 = _rotl(x1, r)
            x1 = x0 ^ x1
        x0 = x0 + _i32(a)
        x1 = x1 + _i32(np.uint32(b) + np.uint32(c))
    return x0, x1


_TINY = np.float32(np.finfo(np.float32).tiny)
_SPAN = np.float32(np.float32(1.0) - _TINY)

CHUNK = 128
N_FULL = 7
TAIL = K - N_FULL * CHUNK
S_PER_STEP = 5
GRID = S_PAD // S_PER_STEP

_NEG_HUGE = np.float32(-1e30)


def _chunk_update(s, base_k, rows, logits_t_ref, classes_ref, carry):
    m, cv = carry
    iota_r = jax.lax.broadcasted_iota(jnp.int32, (rows, B), 0)
    iota_b = jax.lax.broadcasted_iota(jnp.int32, (rows, B), 1)
    flat = (s * (B * K) + base_k) + (iota_b * K + iota_r)
    h0, h1 = _threefry_hash(jnp.zeros((rows, B), jnp.int32), flat)
    bits = h0 ^ h1
    float_bits = jax.lax.shift_right_logical(bits, jnp.int32(9)) | _i32(0x3F800000)
    floats = jax.lax.bitcast_convert_type(float_bits, jnp.float32) - jnp.float32(1.0)
    u = jnp.maximum(_TINY, floats)
    g = -jnp.log(-jnp.log(u))
    val = logits_t_ref[pl.ds(base_k, rows), :] + g
    cm = jnp.max(val, axis=0, keepdims=True)
    ic = jnp.min(jnp.where(val == cm, iota_r, rows), axis=0, keepdims=True)
    cls = classes_ref[pl.ds(base_k, rows), :]
    ccls = jnp.sum(jnp.where(iota_r == ic, cls, jnp.float32(0.0)), axis=0,
                   keepdims=True)
    upd = cm > m
    return jnp.where(upd, cm, m), jnp.where(upd, ccls, cv)


def _sampler_kernel(logits_t_ref, classes_ref, out_ref):
    step = pl.program_id(0)

    def sample_body(sl, _):
        s = step * S_PER_STEP + sl

        def chunk_body(c, carry):
            return _chunk_update(
                s, c * CHUNK, CHUNK, logits_t_ref, classes_ref, carry
            )

        m0 = jnp.full((1, B), -jnp.inf, jnp.float32)
        bc0 = jnp.zeros((1, B), jnp.float32)
        carry = jax.lax.fori_loop(0, N_FULL, chunk_body, (m0, bc0), unroll=7)
        _, bc = _chunk_update(
            s, N_FULL * CHUNK, TAIL, logits_t_ref, classes_ref, carry
        )
        out_ref[pl.ds(sl, 1), 0, :] = bc
        return 0

    jax.lax.fori_loop(0, S_PER_STEP, sample_body, 0, unroll=S_PER_STEP)


def kernel(logits, classes):
    logits_t = logits.T
    classes_col = classes.reshape(K, 1)
    out = pl.pallas_call(
        _sampler_kernel,
        grid=(GRID,),
        in_specs=[
            pl.BlockSpec((K, B), lambda i: (0, 0)),
            pl.BlockSpec((K, 1), lambda i: (0, 0)),
        ],
        out_specs=pl.BlockSpec((S_PER_STEP, 1, B), lambda i: (i, 0, 0)),
        out_shape=jax.ShapeDtypeStruct((S_PAD, 1, B), jnp.float32),
        compiler_params=pltpu.CompilerParams(
            dimension_semantics=("parallel",),
        ),
    )(logits_t, classes_col)
    return out[:S, 0, :].T.reshape(B, 1, 1, S)


# speedup vs baseline: 1.0288x; 1.0030x over previous
import functools

import numpy as np
import jax
import jax.numpy as jnp
from jax import lax
from jax.experimental import pallas as pl
from jax.experimental.pallas import tpu as pltpu
from jax.experimental.pallas import tpu_sc as plsc

S = 250
B = 128
K = 1000
S_PAD = 250


def _np_threefry2x32(k1, k2, x0, x1):
    def rotl(x, d):
        return ((x << np.uint32(d)) | (x >> np.uint32(32 - d))).astype(np.uint32)

    ks0, ks1 = np.uint32(k1), np.uint32(k2)
    ks2 = np.uint32(ks0 ^ ks1 ^ np.uint32(0x1BD11BDA))
    rots = ([13, 15, 26, 6], [17, 29, 16, 24])
    sched = [(ks1, ks2, 1), (ks2, ks0, 2), (ks0, ks1, 3), (ks1, ks2, 4), (ks2, ks0, 5)]
    x0 = np.uint32(x0 + ks0)
    x1 = np.uint32(x1 + ks1)
    for i, (a, b, c) in enumerate(sched):
        for r in rots[i % 2]:
            x0 = np.uint32(x0 + x1)
            x1 = rotl(x1, r)
            x1 = np.uint32(x0 ^ x1)
        x0 = np.uint32(x0 + a)
        x1 = np.uint32(x1 + b + np.uint32(c))
    return x0, x1


_GK1, _GK2 = _np_threefry2x32(0, 0, np.uint32(0), np.uint32(123))
_KS = (int(_GK1), int(_GK2), int(np.uint32(_GK1 ^ _GK2 ^ np.uint32(0x1BD11BDA))))


def _i32(v):
    return jnp.int32(np.int32(np.uint32(v)))


def _rotl(x, d):
    return jax.lax.shift_left(x, jnp.int32(d)) | jax.lax.shift_right_logical(
        x, jnp.int32(32 - d)
    )


def _threefry_hash(x0, x1):
    rots = ([13, 15, 26, 6], [17, 29, 16, 24])
    sched = [
        (_KS[1], _KS[2], 1),
        (_KS[2], _KS[0], 2),
        (_KS[0], _KS[1], 3),
        (_KS[1], _KS[2], 4),
        (_KS[2], _KS[0], 5),
    ]
    x0 = x0 + _i32(_KS[0])
    x1 = x1 + _i32(_KS[1])
    for i, (a, b, c) in enumerate(sched):
        for r in rots[i % 2]:
            x0 = x0 + x1
            x1 = _rotl(x1, r)
            x1 = x0 ^ x1
        x0 = x0 + _i32(a)
        x1 = x1 + _i32(np.uint32(b) + np.uint32(c))
    return x0, x1


_TINY = np.float32(np.finfo(np.float32).tiny)
_SPAN = np.float32(np.float32(1.0) - _TINY)

CHUNK = 128
N_FULL = 7
TAIL = K - N_FULL * CHUNK
S_PER_STEP = 5
GRID = S_PAD // S_PER_STEP

_NEG_HUGE = np.float32(-1e30)


def _chunk_update(s, base_k, rows, logits_t_ref, classes_ref, carry):
    m, cv = carry
    iota_r = jax.lax.broadcasted_iota(jnp.int32, (rows, B), 0)
    iota_b = jax.lax.broadcasted_iota(jnp.int32, (rows, B), 1)
    flat = (s * (B * K) + base_k) + (iota_b * K + iota_r)
    h0, h1 = _threefry_hash(jnp.zeros((rows, B), jnp.int32), flat)
    bits = h0 ^ h1
    float_bits = jax.lax.shift_right_logical(bits, jnp.int32(9)) | _i32(0x3F800000)
    floats = jax.lax.bitcast_convert_type(float_bits, jnp.float32) - jnp.float32(1.0)
    u = jnp.maximum(_TINY, floats)
    g = -jnp.log(-jnp.log(u))
    val = logits_t_ref[pl.ds(base_k, rows), :] + g
    cm = jnp.max(val, axis=0, keepdims=True)
    ic = jnp.min(jnp.where(val == cm, iota_r, rows), axis=0, keepdims=True)
    cls = classes_ref[pl.ds(base_k, rows), :]
    ccls = jnp.sum(jnp.where(iota_r == ic, cls, jnp.float32(0.0)), axis=0,
                   keepdims=True)
    upd = cm > m
    return jnp.where(upd, cm, m), jnp.where(upd, ccls, cv)


def _sampler_kernel(logits_t_ref, classes_ref, out_ref):
    def sample_body(s, _):
        def chunk_body(c, carry):
            return _chunk_update(
                s, c * CHUNK, CHUNK, logits_t_ref, classes_ref, carry
            )

        m0 = jnp.full((1, B), -jnp.inf, jnp.float32)
        bc0 = jnp.zeros((1, B), jnp.float32)
        carry = jax.lax.fori_loop(0, N_FULL, chunk_body, (m0, bc0), unroll=7)
        _, bc = _chunk_update(
            s, N_FULL * CHUNK, TAIL, logits_t_ref, classes_ref, carry
        )
        out_ref[pl.ds(s, 1), 0, :] = bc
        return 0

    jax.lax.fori_loop(0, S, sample_body, 0, unroll=S_PER_STEP)


def kernel(logits, classes):
    logits_t = logits.T
    classes_col = classes.reshape(K, 1)
    out = pl.pallas_call(
        _sampler_kernel,
        grid=(1,),
        in_specs=[
            pl.BlockSpec((K, B), lambda i: (0, 0)),
            pl.BlockSpec((K, 1), lambda i: (0, 0)),
        ],
        out_specs=pl.BlockSpec((S_PAD, 1, B), lambda i: (0, 0, 0)),
        out_shape=jax.ShapeDtypeStruct((S_PAD, 1, B), jnp.float32),
    )(logits_t, classes_col)
    return out[:S, 0, :].T.reshape(B, 1, 1, S)
